# baseline (speedup 1.0000x reference)
import functools

import jax
import jax.numpy as jnp
from jax import lax
from jax.experimental import pallas as pl
from jax.experimental.pallas import tpu as pltpu
from jax.experimental.pallas import tpu_sc as plsc

N = 1_000_000
K = 16_384
L = 16
NC, NS = 2, 16
NW = NC * NS
SHARD = 31_256
LAST_BASE = N - SHARD
CHUNKS = K // L


_mesh = plsc.VectorSubcoreMesh(core_axis_name="c", subcore_axis_name="s")


@functools.partial(
    pl.kernel,
    mesh=_mesh,
    out_type=jax.ShapeDtypeStruct((N,), jnp.float32),
    scratch_types=[
        pltpu.VMEM((SHARD,), jnp.float32),
        pltpu.VMEM((K,), jnp.int32),
        pltpu.VMEM((K,), jnp.float32),
        pltpu.VMEM_SHARED((K,), jnp.int32),
        pltpu.VMEM_SHARED((K,), jnp.float32),
        pltpu.SemaphoreType.DMA,
        pltpu.SemaphoreType.DMA,
        pltpu.SemaphoreType.DMA,
    ],
    compiler_params=pltpu.CompilerParams(needs_layout_passes=False),
)
def _scatter_set(in_hbm, idx_hbm, val_hbm, out_hbm, shard_v, idx_v, val_v,
                 idx_sh, val_sh, sem, sem2, sem3):
    cid = lax.axis_index("c")
    sid = lax.axis_index("s")
    wid = sid * NC + cid
    base = jnp.where(wid == NW - 1, LAST_BASE, wid * SHARD)

    shard_cpy = pltpu.async_copy(in_hbm.at[pl.ds(base, SHARD)], shard_v, sem)

    kslice = K // NS
    off = sid * kslice
    sicpy = pltpu.async_copy(idx_hbm.at[pl.ds(off, kslice)],
                             idx_sh.at[pl.ds(off, kslice)], sem2)
    svcpy = pltpu.async_copy(val_hbm.at[pl.ds(off, kslice)],
                             val_sh.at[pl.ds(off, kslice)], sem2)
    sicpy.wait()
    svcpy.wait()
    plsc.subcore_barrier()
    H = K // 2
    icpy0 = pltpu.async_copy(idx_sh.at[pl.ds(0, H)], idx_v.at[pl.ds(0, H)],
                             sem2)
    vcpy0 = pltpu.async_copy(val_sh.at[pl.ds(0, H)], val_v.at[pl.ds(0, H)],
                             sem2)
    icpy1 = pltpu.async_copy(idx_sh.at[pl.ds(H, H)], idx_v.at[pl.ds(H, H)],
                             sem3)
    vcpy1 = pltpu.async_copy(val_sh.at[pl.ds(H, H)], val_v.at[pl.ds(H, H)],
                             sem3)
    icpy0.wait()
    vcpy0.wait()
    shard_cpy.wait()

    U = 16

    def body(g, carry):
        s0 = g * (L * U)
        rels, vvs, ms = [], [], []
        for k in range(U):
            iv = idx_v[pl.ds(s0 + k * L, L)]
            vvs.append(val_v[pl.ds(s0 + k * L, L)])
            rel = iv - base
            rels.append(rel)
            ms.append(plsc.bitcast(rel, jnp.uint32) < jnp.uint32(SHARD))
        lasts = [plsc.scan_count(rels[k], ms[k])[1] for k in range(U)]
        for k in range(U):
            plsc.store_scatter(shard_v, [rels[k]], vvs[k],
                               mask=lasts[k] & ms[k])
        return carry

    GROUPS = CHUNKS // U
    lax.fori_loop(0, GROUPS // 2, body, 0, unroll=1)
    icpy1.wait()
    vcpy1.wait()
    lax.fori_loop(GROUPS // 2, GROUPS, body, 0, unroll=1)

    pltpu.sync_copy(shard_v, out_hbm.at[pl.ds(base, SHARD)])


def kernel(input, index, value):
    return _scatter_set(input, index.astype(jnp.int32), value)


# speedup vs baseline: 1.0041x; 1.0041x over previous
import functools

import jax
import jax.numpy as jnp
from jax import lax
from jax.experimental import pallas as pl
from jax.experimental.pallas import tpu as pltpu
from jax.experimental.pallas import tpu_sc as plsc

N = 1_000_000
K = 16_384
L = 16
NC, NS = 2, 16
NW = NC * NS
SHARD = 31_256
LAST_BASE = N - SHARD
CHUNKS = K // L


_mesh = plsc.VectorSubcoreMesh(core_axis_name="c", subcore_axis_name="s")


@functools.partial(
    pl.kernel,
    mesh=_mesh,
    out_type=jax.ShapeDtypeStruct((N,), jnp.float32),
    scratch_types=[
        pltpu.VMEM((SHARD,), jnp.float32),
        pltpu.VMEM((K,), jnp.int32),
        pltpu.VMEM((K,), jnp.float32),
        pltpu.VMEM_SHARED((K,), jnp.int32),
        pltpu.VMEM_SHARED((K,), jnp.float32),
        pltpu.SemaphoreType.DMA,
        pltpu.SemaphoreType.DMA,
        pltpu.SemaphoreType.DMA,
    ],
    compiler_params=pltpu.CompilerParams(needs_layout_passes=False),
)
def _scatter_set(in_hbm, idx_hbm, val_hbm, out_hbm, shard_v, idx_v, val_v,
                 idx_sh, val_sh, sem, sem2, sem3):
    cid = lax.axis_index("c")
    sid = lax.axis_index("s")
    wid = sid * NC + cid
    base = jnp.where(wid == NW - 1, LAST_BASE, wid * SHARD)

    shard_cpy = pltpu.async_copy(in_hbm.at[pl.ds(base, SHARD)], shard_v, sem)

    kslice = K // NS
    off = sid * kslice
    sicpy = pltpu.async_copy(idx_hbm.at[pl.ds(off, kslice)],
                             idx_sh.at[pl.ds(off, kslice)], sem2)
    svcpy = pltpu.async_copy(val_hbm.at[pl.ds(off, kslice)],
                             val_sh.at[pl.ds(off, kslice)], sem2)
    sicpy.wait()
    svcpy.wait()
    plsc.subcore_barrier()
    icpy = pltpu.async_copy(idx_sh, idx_v, sem2)
    vcpy = pltpu.async_copy(val_sh, val_v, sem3)
    icpy.wait()
    vcpy.wait()
    shard_cpy.wait()

    U = 16

    def body(g, carry):
        s0 = g * (L * U)
        rels, vvs, ms = [], [], []
        for k in range(U):
            iv = idx_v[pl.ds(s0 + k * L, L)]
            vvs.append(val_v[pl.ds(s0 + k * L, L)])
            rel = iv - base
            rels.append(rel)
            ms.append(plsc.bitcast(rel, jnp.uint32) < jnp.uint32(SHARD))
        lasts = [plsc.scan_count(rels[k], ms[k])[1] for k in range(U)]
        for k in range(U):
            plsc.store_scatter(shard_v, [rels[k]], vvs[k],
                               mask=lasts[k] & ms[k])
        return carry

    lax.fori_loop(0, CHUNKS // U, body, 0, unroll=1)

    pltpu.sync_copy(shard_v, out_hbm.at[pl.ds(base, SHARD)])


def kernel(input, index, value):
    return _scatter_set(input, index.astype(jnp.int32), value)
